# baseline (speedup 1.0000x reference)
import functools

import numpy as np
import jax
import jax.numpy as jnp
from jax.experimental import pallas as pl
from jax.experimental.pallas import tpu as pltpu

_K = 3
_D = 100
_M1 = 0.05
_M2 = 0.25
_REG = 10.0


def _band_matrix():
    w = np.zeros((_K, _K * _D), np.float32)
    for p in range(_K):
        w[p, p * _D:(p + 1) * _D] = 1.0
    return w


def _body(x_ref, w_ref, o_ref, *, grid, ncols, block, scale):
    i = pl.program_id(0)
    x = x_ref[...]
    s = jnp.sum(x)

    @pl.when(i == 0)
    def _init():
        o_ref[0, 0] = 0.0

    o_ref[0, 0] += s

    @pl.when(i == grid - 1)
    def _fin():
        o_ref[0, 0] *= scale


def kernel(user_ids, pos_ids, neg_ids, user_emb, item_emb):
    n, kd = user_emb.shape
    xt = user_emb.T
    block = 9216
    grid = (n + block - 1) // block
    scale = 2.0 * _REG / (n * _K * _K)
    wmat = jnp.asarray(_band_matrix(), dtype=jnp.float32)
    out = pl.pallas_call(
        functools.partial(_body, grid=grid, ncols=n, block=block,
                          scale=scale),
        grid=(grid,),
        in_specs=[
            pl.BlockSpec((kd, block), lambda i: (0, i)),
            pl.BlockSpec(wmat.shape, lambda i: (0, 0)),
        ],
        out_specs=pl.BlockSpec((1, 1), lambda i: (0, 0),
                               memory_space=pltpu.SMEM),
        out_shape=jax.ShapeDtypeStruct((1, 1), jnp.float32),
    )(xt, wmat)
    return out[0, 0]


# speedup vs baseline: 1.1443x; 1.1443x over previous
import functools

import numpy as np
import jax
import jax.numpy as jnp
from jax.experimental import pallas as pl
from jax.experimental.pallas import tpu as pltpu

_K = 3
_D = 100
_M1 = 0.05
_M2 = 0.25
_REG = 10.0


def _band_matrix():
    w = np.zeros((_K, _K * _D), np.float32)
    for p in range(_K):
        w[p, p * _D:(p + 1) * _D] = 1.0
    return w


def _body(x_ref, w_ref, o_ref, *, grid, ncols, block, scale):
    i = pl.program_id(0)
    x = x_ref[...]
    r = jnp.roll(x, -_D, axis=0)
    z = (x - r) ** 2
    d = jax.lax.dot_general(w_ref[...], z, (((1,), (0,)), ((), ())),
                            preferred_element_type=jnp.float32)
    h = jnp.maximum(_M1 - d, 0.0) + jnp.maximum(d - _M2, 0.0)
    col = jax.lax.broadcasted_iota(jnp.int32, h.shape, 1) + i * block
    s = jnp.sum(jnp.where(col < ncols, h, 0.0))

    @pl.when(i == 0)
    def _init():
        o_ref[0, 0] = 0.0

    o_ref[0, 0] += s

    @pl.when(i == grid - 1)
    def _fin():
        o_ref[0, 0] *= scale


def kernel(user_ids, pos_ids, neg_ids, user_emb, item_emb):
    n, kd = user_emb.shape
    xt = user_emb.T
    block = 9216
    grid = (n + block - 1) // block
    scale = 2.0 * _REG / (n * _K * _K)
    wmat = jnp.asarray(_band_matrix(), dtype=jnp.float32)
    out = pl.pallas_call(
        functools.partial(_body, grid=grid, ncols=n, block=block,
                          scale=scale),
        grid=(grid,),
        in_specs=[
            pl.BlockSpec((kd, block), lambda i: (0, i)),
            pl.BlockSpec(wmat.shape, lambda i: (0, 0)),
        ],
        out_specs=pl.BlockSpec((1, 1), lambda i: (0, 0),
                               memory_space=pltpu.SMEM),
        out_shape=jax.ShapeDtypeStruct((1, 1), jnp.float32),
    )(xt, wmat)
    return out[0, 0]
